# Initial kernel scaffold; baseline (speedup 1.0000x reference)
import jax
import jax.numpy as jnp
from jax.experimental import pallas as pl


def kernel(x, edge_index, conv1_W, conv1_b, conv2_W, conv2_b, emb_W, emb_b, deg_W1, deg_b1, deg_W2, deg_b2, gs_W1, gs_b1, gs_W2, gs_b2, fin_W, fin_b):
    raise NotImplementedError("write your pallas kernel here")


# speedup vs baseline: 14.0096x; 14.0096x over previous
import functools

import numpy as np
import jax
import jax.numpy as jnp
from jax import lax
from jax.experimental import pallas as pl
from jax.experimental.pallas import tpu as pltpu
from jax.experimental.pallas import tpu_sc as plsc

_NC = 2
_NS = 16
_CHUNK = 80


def _make_degree_kernel(n, e):
    per_w = e // (_NC * _NS)
    n_chunks = per_w // _CHUNK
    mesh = plsc.VectorSubcoreMesh(core_axis_name="c", subcore_axis_name="s")

    @functools.partial(
        pl.kernel,
        out_type=(
            jax.ShapeDtypeStruct((_NC, n), jnp.float32),
            jax.ShapeDtypeStruct((_NC, n), jnp.float32),
        ),
        mesh=mesh,
        scratch_types=[
            pltpu.VMEM((_CHUNK,), jnp.int32),
            pltpu.VMEM((_CHUNK,), jnp.int32),
            pltpu.VMEM((_CHUNK,), jnp.float32),
            pltpu.VMEM_SHARED((n,), jnp.float32),
            pltpu.VMEM_SHARED((n,), jnp.float32),
        ],
    )
    def deg_kernel(row_h, col_h, zeros_h, outc_h, outr_h,
                   ridx, cidx, ones_v, acc_c, acc_r):
        cid = lax.axis_index("c")
        sid = lax.axis_index("s")

        def fill(i, c):
            ones_v[pl.ds(i * 16, 16)] = jnp.ones((16,), jnp.float32)
            return c
        lax.fori_loop(0, _CHUNK // 16, fill, 0)

        @pl.when(sid == 0)
        def _():
            pltpu.sync_copy(zeros_h, acc_c)
            pltpu.sync_copy(zeros_h, acc_r)
        plsc.subcore_barrier()

        base = (cid * _NS + sid) * per_w

        def body(i, c):
            off = base + i * _CHUNK
            pltpu.sync_copy(row_h.at[pl.ds(off, _CHUNK)], ridx)
            pltpu.sync_copy(col_h.at[pl.ds(off, _CHUNK)], cidx)
            pltpu.sync_copy(ones_v, acc_c.at[cidx], add=True)
            pltpu.sync_copy(ones_v, acc_r.at[ridx], add=True)
            return c
        lax.fori_loop(0, n_chunks, body, 0)
        plsc.subcore_barrier()

        @pl.when(sid == 0)
        def _():
            pltpu.sync_copy(acc_c, outc_h.at[cid])
            pltpu.sync_copy(acc_r, outr_h.at[cid])

    return deg_kernel


def _make_mp_kernel(n, e, d):
    per_w = e // (_NC * _NS)
    n_chunks = per_w // _CHUNK
    mesh = plsc.VectorSubcoreMesh(core_axis_name="c", subcore_axis_name="s")

    @functools.partial(
        pl.kernel,
        out_type=jax.ShapeDtypeStruct((_NC, n, d), jnp.float32),
        mesh=mesh,
        compiler_params=pltpu.CompilerParams(use_tc_tiling_on_sc=False),
        scratch_types=[
            pltpu.VMEM((_CHUNK,), jnp.int32),
            pltpu.VMEM((_CHUNK,), jnp.int32),
            pltpu.VMEM((_CHUNK, d), jnp.float32),
            pltpu.VMEM_SHARED((n, d), jnp.float32),
            pltpu.SemaphoreType.DMA,
        ],
    )
    def mp_kernel(g_h, row_h, col_h, zeros_h, out_h,
                  ridx, cidx, msg, acc, sem):
        cid = lax.axis_index("c")
        sid = lax.axis_index("s")

        @pl.when(sid == 0)
        def _():
            pltpu.sync_copy(zeros_h, acc)
        plsc.subcore_barrier()

        base = (cid * _NS + sid) * per_w

        def body(i, c):
            off = base + i * _CHUNK
            pltpu.sync_copy(row_h.at[pl.ds(off, _CHUNK)], ridx)
            pltpu.sync_copy(col_h.at[pl.ds(off, _CHUNK)], cidx)
            pltpu.async_copy(g_h.at[ridx], msg, sem).wait()
            pltpu.sync_copy(msg, acc.at[cidx], add=True)
            return c
        lax.fori_loop(0, n_chunks, body, 0)
        plsc.subcore_barrier()

        @pl.when(sid == 0)
        def _():
            pltpu.sync_copy(acc, out_h.at[cid])

    return mp_kernel


def _prep_body(degc_ref, degr_ref, x_ref, w1_ref, dw1_ref, db1_ref,
               dw2_ref, db2_ref, g1_ref, dinv_ref, demb_ref):
    degc = degc_ref[...]
    dinv = lax.rsqrt(degc[0] + degc[1] + 1.0)
    dinv_ref[...] = dinv
    g1_ref[...] = jnp.dot(x_ref[...], w1_ref[...],
                          preferred_element_type=jnp.float32) * dinv[:, None]

    degr = degr_ref[...]
    outdeg = degr[0] + degr[1]
    n = outdeg.shape[0]
    demb = db2_ref[...][None, :]
    for j in range(8):
        sj = jnp.sum(jax.nn.relu(outdeg * dw1_ref[0, j] + db1_ref[j])) / n
        demb = demb + sj * dw2_ref[j:j + 1, :]
    demb_ref[...] = demb


def _mid_body(s_ref, g1_ref, dinv_ref, b1_ref, w2_ref, g2_ref):
    dinv = dinv_ref[...]
    h1 = jax.nn.relu((s_ref[0] + s_ref[1] + g1_ref[...]) * dinv[:, None]
                     + b1_ref[...][None, :])
    g2_ref[...] = jnp.dot(h1, w2_ref[...],
                          preferred_element_type=jnp.float32) * dinv[:, None]


def _fin_body(s_ref, g2_ref, dinv_ref, b2_ref, embw_ref, embb_ref, demb_ref,
              gsw1_ref, gsb1_ref, gsw2_ref, gsb2_ref, finw_ref, finb_ref,
              stats_ref, out_ref):
    dinv = dinv_ref[...]
    h2 = jax.nn.relu((s_ref[0] + s_ref[1] + g2_ref[...]) * dinv[:, None]
                     + b2_ref[...][None, :])
    hid = h2.shape[1]
    p_sum = jnp.sum(h2, axis=0, keepdims=True)
    p_mean = p_sum / h2.shape[0]
    p_max = jnp.max(h2, axis=0, keepdims=True)
    embw = embw_ref[...]
    dot = functools.partial(jnp.dot, preferred_element_type=jnp.float32)
    ge = jax.nn.relu(dot(p_mean, embw[0:hid])
                     + dot(p_sum, embw[hid:2 * hid])
                     + dot(p_max, embw[2 * hid:3 * hid])
                     + embb_ref[...][None, :])
    a1 = jax.nn.relu(dot(stats_ref[...], gsw1_ref[...]) + gsb1_ref[...][None, :])
    gse = dot(a1, gsw2_ref[...]) + gsb2_ref[...][None, :]
    finw = finw_ref[...]
    obs = (dot(ge, finw[0:64]) + dot(demb_ref[...], finw[64:72])
           + dot(gse, finw[72:88]) + finb_ref[...][None, :])
    out_ref[...] = obs


def _tc_call(body, out_shape, *args):
    return pl.pallas_call(body, out_shape=out_shape)(*args)


def kernel(x, edge_index, conv1_W, conv1_b, conv2_W, conv2_b, emb_W, emb_b,
           deg_W1, deg_b1, deg_W2, deg_b2, gs_W1, gs_b1, gs_W2, gs_b2,
           fin_W, fin_b):
    n, in_ch = x.shape
    e = edge_index.shape[1]
    hid = conv1_W.shape[1]
    row = edge_index[0]
    col = edge_index[1]

    zeros_n = jnp.zeros((n,), jnp.float32)
    zeros_2d = jnp.zeros((n, hid), jnp.float32)

    graph_size = n / 100.0
    edge_density = (e / 2.0) / (n * (n - 1) / 2.0)
    avg_path_length = 1.0 / (edge_density + 1e-06)
    stats = jnp.asarray(np.array(
        [[graph_size, edge_density, edge_density, avg_path_length, 1.0]],
        dtype=np.float32))

    deg_k = _make_degree_kernel(n, e)
    mp_k = _make_mp_kernel(n, e, hid)

    degc, degr = deg_k(row, col, zeros_n)

    g1, dinv, demb = _tc_call(
        _prep_body,
        (jax.ShapeDtypeStruct((n, hid), jnp.float32),
         jax.ShapeDtypeStruct((n,), jnp.float32),
         jax.ShapeDtypeStruct((1, 8), jnp.float32)),
        degc, degr, x, conv1_W, deg_W1, deg_b1, deg_W2, deg_b2)

    s1 = mp_k(g1, row, col, zeros_2d)

    g2 = _tc_call(
        _mid_body,
        jax.ShapeDtypeStruct((n, hid), jnp.float32),
        s1, g1, dinv, conv1_b, conv2_W)

    s2 = mp_k(g2, row, col, zeros_2d)

    obs = _tc_call(
        _fin_body,
        jax.ShapeDtypeStruct((1, 64), jnp.float32),
        s2, g2, dinv, conv2_b, emb_W, emb_b, demb,
        gs_W1, gs_b1, gs_W2, gs_b2, fin_W, fin_b, stats)

    return obs.reshape(-1)


# speedup vs baseline: 18.7154x; 1.3359x over previous
import functools

import numpy as np
import jax
import jax.numpy as jnp
from jax import lax
from jax.experimental import pallas as pl
from jax.experimental.pallas import tpu as pltpu
from jax.experimental.pallas import tpu_sc as plsc

_NC = 2
_NS = 16
_NW = _NC * _NS
_CB = 128
_CE = _CB
_PAD = 8


def _make_degree_kernel(n, nch):
    npad = n + _PAD
    mesh = plsc.VectorSubcoreMesh(core_axis_name="c", subcore_axis_name="s")

    @functools.partial(
        pl.kernel,
        out_type=(
            jax.ShapeDtypeStruct((_NC, n), jnp.float32),
            jax.ShapeDtypeStruct((_NC, n), jnp.float32),
        ),
        mesh=mesh,
        compiler_params=pltpu.CompilerParams(use_tc_tiling_on_sc=False),
        scratch_types=[
            pltpu.VMEM((nch, _CB), jnp.int32),
            pltpu.VMEM((nch, _CB), jnp.int32),
            pltpu.VMEM((_CB,), jnp.float32),
            pltpu.VMEM_SHARED((npad,), jnp.float32),
            pltpu.VMEM_SHARED((npad,), jnp.float32),
            pltpu.SemaphoreType.DMA,
        ],
    )
    def deg_kernel(row_h, col_h, zeros_h, outc_h, outr_h,
                   ridx, cidx, ones_v, acc_c, acc_r, ssem):
        cid = lax.axis_index("c")
        sid = lax.axis_index("s")
        wid = cid * _NS + sid

        def fill(i, c):
            ones_v[pl.ds(i * 16, 16)] = jnp.ones((16,), jnp.float32)
            return c
        lax.fori_loop(0, _CB // 16, fill, 0)

        pltpu.sync_copy(row_h.at[wid], ridx)
        pltpu.sync_copy(col_h.at[wid], cidx)

        @pl.when(sid == 0)
        def _():
            pltpu.sync_copy(zeros_h, acc_c)
            pltpu.sync_copy(zeros_h, acc_r)
        plsc.subcore_barrier()

        def body(i, c):
            pltpu.async_copy(ones_v, acc_c.at[cidx.at[i]], ssem, add=True)
            pltpu.sync_copy(ones_v, acc_r.at[ridx.at[i]], add=True)
            return c
        lax.fori_loop(0, nch, body, 0)

        def drain(i, c):
            pltpu.make_async_copy(ones_v, acc_c.at[cidx.at[0]], ssem).wait()
            return c
        lax.fori_loop(0, nch, drain, 0)
        plsc.subcore_barrier()

        @pl.when(sid == 0)
        def _():
            pltpu.sync_copy(acc_c.at[pl.ds(0, n)], outc_h.at[cid])
            pltpu.sync_copy(acc_r.at[pl.ds(0, n)], outr_h.at[cid])

    return deg_kernel


def _make_mp_kernel(n, nch, d):
    npad = n + _PAD
    mesh = plsc.VectorSubcoreMesh(core_axis_name="c", subcore_axis_name="s")
    assert nch % 2 == 0

    @functools.partial(
        pl.kernel,
        out_type=jax.ShapeDtypeStruct((_NC, n, d), jnp.float32),
        mesh=mesh,
        compiler_params=pltpu.CompilerParams(use_tc_tiling_on_sc=False),
        scratch_types=[
            pltpu.VMEM((nch, _CB), jnp.int32),
            pltpu.VMEM((nch, _CB), jnp.int32),
            pltpu.VMEM((_CB, d), jnp.float32),
            pltpu.VMEM((_CB, d), jnp.float32),
            pltpu.VMEM_SHARED((npad, d), jnp.float32),
            pltpu.SemaphoreType.DMA,
            pltpu.SemaphoreType.DMA,
        ],
    )
    def mp_kernel(g_h, row_h, col_h, zeros_h, out_h,
                  ridx, cidx, msg_a, msg_b, acc, gsem_a, gsem_b):
        cid = lax.axis_index("c")
        sid = lax.axis_index("s")
        wid = cid * _NS + sid

        pltpu.sync_copy(row_h.at[wid], ridx)
        pltpu.sync_copy(col_h.at[wid], cidx)

        @pl.when(sid == 0)
        def _():
            pltpu.sync_copy(zeros_h, acc)
        plsc.subcore_barrier()

        pltpu.async_copy(g_h.at[ridx.at[0]], msg_a, gsem_a)
        pltpu.async_copy(g_h.at[ridx.at[1]], msg_b, gsem_b)

        def body(p, c):
            i = 2 * p
            pltpu.make_async_copy(g_h.at[ridx.at[0]], msg_a, gsem_a).wait()
            pltpu.sync_copy(msg_a, acc.at[cidx.at[i]], add=True)

            @pl.when(i + 2 < nch)
            def _():
                pltpu.async_copy(g_h.at[ridx.at[i + 2]], msg_a, gsem_a)

            pltpu.make_async_copy(g_h.at[ridx.at[0]], msg_b, gsem_b).wait()
            pltpu.sync_copy(msg_b, acc.at[cidx.at[i + 1]], add=True)

            @pl.when(i + 3 < nch)
            def _():
                pltpu.async_copy(g_h.at[ridx.at[i + 3]], msg_b, gsem_b)
            return c
        lax.fori_loop(0, nch // 2, body, 0)
        plsc.subcore_barrier()

        @pl.when(sid == 0)
        def _():
            pltpu.sync_copy(acc.at[pl.ds(0, n)], out_h.at[cid])

    return mp_kernel


def _make_prep_body(pad_corr):
    def _prep_body(degc_ref, degr_ref, x_ref, w1_ref, dw1_ref, db1_ref,
                   dw2_ref, db2_ref, g1_ref, dinv_ref, demb_ref):
        degc = degc_ref[...]
        dinv = lax.rsqrt(degc[0] + degc[1] + 1.0)
        dinv_ref[...] = dinv
        g1_ref[...] = jnp.dot(x_ref[...], w1_ref[...],
                              preferred_element_type=jnp.float32) * dinv[:, None]

        degr = degr_ref[...]
        outdeg = degr[0] + degr[1]
        n = outdeg.shape[0]
        lane = lax.broadcasted_iota(jnp.int32, (n,), 0)
        outdeg = outdeg - jnp.where(lane == 0, pad_corr, 0.0)
        demb = db2_ref[...][None, :]
        for j in range(8):
            sj = jnp.sum(jax.nn.relu(outdeg * dw1_ref[0, j] + db1_ref[j])) / n
            demb = demb + sj * dw2_ref[j:j + 1, :]
        demb_ref[...] = demb
    return _prep_body


def _mid_body(s_ref, g1_ref, dinv_ref, b1_ref, w2_ref, g2_ref):
    dinv = dinv_ref[...]
    h1 = jax.nn.relu((s_ref[0] + s_ref[1] + g1_ref[...]) * dinv[:, None]
                     + b1_ref[...][None, :])
    g2_ref[...] = jnp.dot(h1, w2_ref[...],
                          preferred_element_type=jnp.float32) * dinv[:, None]


def _fin_body(s_ref, g2_ref, dinv_ref, b2_ref, embw_ref, embb_ref, demb_ref,
              gsw1_ref, gsb1_ref, gsw2_ref, gsb2_ref, finw_ref, finb_ref,
              stats_ref, out_ref):
    dinv = dinv_ref[...]
    h2 = jax.nn.relu((s_ref[0] + s_ref[1] + g2_ref[...]) * dinv[:, None]
                     + b2_ref[...][None, :])
    hid = h2.shape[1]
    p_sum = jnp.sum(h2, axis=0, keepdims=True)
    p_mean = p_sum / h2.shape[0]
    p_max = jnp.max(h2, axis=0, keepdims=True)
    embw = embw_ref[...]
    dot = functools.partial(jnp.dot, preferred_element_type=jnp.float32)
    ge = jax.nn.relu(dot(p_mean, embw[0:hid])
                     + dot(p_sum, embw[hid:2 * hid])
                     + dot(p_max, embw[2 * hid:3 * hid])
                     + embb_ref[...][None, :])
    a1 = jax.nn.relu(dot(stats_ref[...], gsw1_ref[...]) + gsb1_ref[...][None, :])
    gse = dot(a1, gsw2_ref[...]) + gsb2_ref[...][None, :]
    finw = finw_ref[...]
    obs = (dot(ge, finw[0:64]) + dot(demb_ref[...], finw[64:72])
           + dot(gse, finw[72:88]) + finb_ref[...][None, :])
    out_ref[...] = obs


def _tc_call(body, out_shape, *args):
    return pl.pallas_call(body, out_shape=out_shape)(*args)


def kernel(x, edge_index, conv1_W, conv1_b, conv2_W, conv2_b, emb_W, emb_b,
           deg_W1, deg_b1, deg_W2, deg_b2, gs_W1, gs_b1, gs_W2, gs_b2,
           fin_W, fin_b):
    n, in_ch = x.shape
    e = edge_index.shape[1]
    hid = conv1_W.shape[1]

    per_w = e // _NW
    nch = -(-per_w // _CE)
    if nch % 2:
        nch += 1
    quota = nch * _CE
    pad = quota - per_w

    row2 = edge_index[0].reshape(_NW, per_w)
    col2 = edge_index[1].reshape(_NW, per_w)
    rowp = jnp.pad(row2, ((0, 0), (0, pad))).reshape(_NW, nch, _CB)
    colp = jnp.pad(col2, ((0, 0), (0, pad)),
                   constant_values=n).reshape(_NW, nch, _CB)

    zeros_n = jnp.zeros((n + _PAD,), jnp.float32)
    zeros_2d = jnp.zeros((n + _PAD, hid), jnp.float32)

    graph_size = n / 100.0
    edge_density = (e / 2.0) / (n * (n - 1) / 2.0)
    avg_path_length = 1.0 / (edge_density + 1e-06)
    stats = jnp.asarray(np.array(
        [[graph_size, edge_density, edge_density, avg_path_length, 1.0]],
        dtype=np.float32))

    deg_k = _make_degree_kernel(n, nch)
    mp_k = _make_mp_kernel(n, nch, hid)

    degc, degr = deg_k(rowp, colp, zeros_n)

    g1, dinv, demb = _tc_call(
        _make_prep_body(float(pad * _NW)),
        (jax.ShapeDtypeStruct((n, hid), jnp.float32),
         jax.ShapeDtypeStruct((n,), jnp.float32),
         jax.ShapeDtypeStruct((1, 8), jnp.float32)),
        degc, degr, x, conv1_W, deg_W1, deg_b1, deg_W2, deg_b2)

    s1 = mp_k(g1, rowp, colp, zeros_2d)

    g2 = _tc_call(
        _mid_body,
        jax.ShapeDtypeStruct((n, hid), jnp.float32),
        s1, g1, dinv, conv1_b, conv2_W)

    s2 = mp_k(g2, rowp, colp, zeros_2d)

    obs = _tc_call(
        _fin_body,
        jax.ShapeDtypeStruct((1, 64), jnp.float32),
        s2, g2, dinv, conv2_b, emb_W, emb_b, demb,
        gs_W1, gs_b1, gs_W2, gs_b2, fin_W, fin_b, stats)

    return obs.reshape(-1)


# speedup vs baseline: 18.9299x; 1.0115x over previous
import functools

import numpy as np
import jax
import jax.numpy as jnp
from jax import lax
from jax.experimental import pallas as pl
from jax.experimental.pallas import tpu as pltpu
from jax.experimental.pallas import tpu_sc as plsc

_NC = 2
_NS = 16
_NW = _NC * _NS
_CB = 128
_CE = _CB
_PAD = 8


def _make_degree_kernel(n, nch):
    npad = n + _PAD
    mesh = plsc.VectorSubcoreMesh(core_axis_name="c", subcore_axis_name="s")

    @functools.partial(
        pl.kernel,
        out_type=(
            jax.ShapeDtypeStruct((_NC, n), jnp.float32),
            jax.ShapeDtypeStruct((_NC, n), jnp.float32),
        ),
        mesh=mesh,
        compiler_params=pltpu.CompilerParams(use_tc_tiling_on_sc=False),
        scratch_types=[
            pltpu.VMEM((nch, _CB), jnp.int32),
            pltpu.VMEM((nch, _CB), jnp.int32),
            pltpu.VMEM((_CB,), jnp.float32),
            pltpu.VMEM_SHARED((npad,), jnp.float32),
            pltpu.VMEM_SHARED((npad,), jnp.float32),
            pltpu.SemaphoreType.DMA,
        ],
    )
    def deg_kernel(row_h, col_h, zeros_h, outc_h, outr_h,
                   ridx, cidx, ones_v, acc_c, acc_r, ssem):
        cid = lax.axis_index("c")
        sid = lax.axis_index("s")
        wid = cid * _NS + sid

        def fill(i, c):
            ones_v[pl.ds(i * 16, 16)] = jnp.ones((16,), jnp.float32)
            return c
        lax.fori_loop(0, _CB // 16, fill, 0)

        pltpu.sync_copy(row_h.at[wid], ridx)
        pltpu.sync_copy(col_h.at[wid], cidx)

        @pl.when(sid == 0)
        def _():
            pltpu.sync_copy(zeros_h, acc_c)
            pltpu.sync_copy(zeros_h, acc_r)
        plsc.subcore_barrier()

        def body(i, c):
            pltpu.async_copy(ones_v, acc_c.at[cidx.at[i]], ssem, add=True)
            pltpu.sync_copy(ones_v, acc_r.at[ridx.at[i]], add=True)
            return c
        lax.fori_loop(0, nch, body, 0)

        def drain(i, c):
            pltpu.make_async_copy(ones_v, acc_c.at[cidx.at[0]], ssem).wait()
            return c
        lax.fori_loop(0, nch, drain, 0)
        plsc.subcore_barrier()

        @pl.when(sid == 0)
        def _():
            pltpu.sync_copy(acc_c.at[pl.ds(0, n)], outc_h.at[cid])
            pltpu.sync_copy(acc_r.at[pl.ds(0, n)], outr_h.at[cid])

    return deg_kernel


def _make_mp_kernel(n, nch, d):
    npad = n + _PAD
    mesh = plsc.VectorSubcoreMesh(core_axis_name="c", subcore_axis_name="s")
    assert nch % 2 == 0

    @functools.partial(
        pl.kernel,
        out_type=jax.ShapeDtypeStruct((_NC, n, d), jnp.float32),
        mesh=mesh,
        compiler_params=pltpu.CompilerParams(use_tc_tiling_on_sc=False),
        scratch_types=[
            pltpu.VMEM((nch, _CB), jnp.int32),
            pltpu.VMEM((nch, _CB), jnp.int32),
            [pltpu.VMEM((_CB, d), jnp.float32) for _ in range(4)],
            pltpu.VMEM_SHARED((npad, d), jnp.float32),
            [pltpu.SemaphoreType.DMA for _ in range(4)],
            [pltpu.SemaphoreType.DMA for _ in range(4)],
        ],
    )
    def mp_kernel(g_h, row_h, col_h, zeros_h, out_h,
                  ridx, cidx, msg, acc, gsem, ssem):
        cid = lax.axis_index("c")
        sid = lax.axis_index("s")
        wid = cid * _NS + sid

        pltpu.sync_copy(row_h.at[wid], ridx)
        pltpu.sync_copy(col_h.at[wid], cidx)

        @pl.when(sid == 0)
        def _():
            pltpu.sync_copy(zeros_h, acc)
        plsc.subcore_barrier()

        def wait_gather(b):
            pltpu.make_async_copy(g_h.at[ridx.at[0]], msg[b], gsem[b]).wait()

        def wait_scatter(b):
            pltpu.make_async_copy(msg[b], acc.at[cidx.at[0]], ssem[b]).wait()

        pltpu.async_copy(g_h.at[ridx.at[0]], msg[0], gsem[0])
        pltpu.async_copy(g_h.at[ridx.at[1]], msg[1], gsem[1])

        assert nch % 4 == 0

        def body(p, c):
            for k in range(4):
                i = 4 * p + k
                b = k
                b2 = (k + 2) % 4
                wait_gather(b)
                pltpu.async_copy(msg[b], acc.at[cidx.at[i]], ssem[b], add=True)

                @pl.when(jnp.logical_and(i >= 2, i + 2 < nch))
                def _():
                    wait_scatter(b2)

                @pl.when(i + 2 < nch)
                def _():
                    pltpu.async_copy(g_h.at[ridx.at[i + 2]], msg[b2], gsem[b2])
            return c
        lax.fori_loop(0, nch // 4, body, 0)
        for j in range(nch - 4, nch):
            wait_scatter(j % 4)
        plsc.subcore_barrier()

        @pl.when(sid == 0)
        def _():
            pltpu.sync_copy(acc.at[pl.ds(0, n)], out_h.at[cid])

    return mp_kernel


def _make_prep_body(pad_corr):
    def _prep_body(degc_ref, degr_ref, x_ref, w1_ref, dw1_ref, db1_ref,
                   dw2_ref, db2_ref, g1_ref, dinv_ref, demb_ref):
        degc = degc_ref[...]
        dinv = lax.rsqrt(degc[0] + degc[1] + 1.0)
        dinv_ref[...] = dinv
        g1_ref[...] = jnp.dot(x_ref[...], w1_ref[...],
                              preferred_element_type=jnp.float32) * dinv[:, None]

        degr = degr_ref[...]
        outdeg = degr[0] + degr[1]
        n = outdeg.shape[0]
        lane = lax.broadcasted_iota(jnp.int32, (n,), 0)
        outdeg = outdeg - jnp.where(lane == 0, pad_corr, 0.0)
        demb = db2_ref[...][None, :]
        for j in range(8):
            sj = jnp.sum(jax.nn.relu(outdeg * dw1_ref[0, j] + db1_ref[j])) / n
            demb = demb + sj * dw2_ref[j:j + 1, :]
        demb_ref[...] = demb
    return _prep_body


def _mid_body(s_ref, g1_ref, dinv_ref, b1_ref, w2_ref, g2_ref):
    dinv = dinv_ref[...]
    h1 = jax.nn.relu((s_ref[0] + s_ref[1] + g1_ref[...]) * dinv[:, None]
                     + b1_ref[...][None, :])
    g2_ref[...] = jnp.dot(h1, w2_ref[...],
                          preferred_element_type=jnp.float32) * dinv[:, None]


def _fin_body(s_ref, g2_ref, dinv_ref, b2_ref, embw_ref, embb_ref, demb_ref,
              gsw1_ref, gsb1_ref, gsw2_ref, gsb2_ref, finw_ref, finb_ref,
              stats_ref, out_ref):
    dinv = dinv_ref[...]
    h2 = jax.nn.relu((s_ref[0] + s_ref[1] + g2_ref[...]) * dinv[:, None]
                     + b2_ref[...][None, :])
    hid = h2.shape[1]
    p_sum = jnp.sum(h2, axis=0, keepdims=True)
    p_mean = p_sum / h2.shape[0]
    p_max = jnp.max(h2, axis=0, keepdims=True)
    embw = embw_ref[...]
    dot = functools.partial(jnp.dot, preferred_element_type=jnp.float32)
    ge = jax.nn.relu(dot(p_mean, embw[0:hid])
                     + dot(p_sum, embw[hid:2 * hid])
                     + dot(p_max, embw[2 * hid:3 * hid])
                     + embb_ref[...][None, :])
    a1 = jax.nn.relu(dot(stats_ref[...], gsw1_ref[...]) + gsb1_ref[...][None, :])
    gse = dot(a1, gsw2_ref[...]) + gsb2_ref[...][None, :]
    finw = finw_ref[...]
    obs = (dot(ge, finw[0:64]) + dot(demb_ref[...], finw[64:72])
           + dot(gse, finw[72:88]) + finb_ref[...][None, :])
    out_ref[...] = obs


def _tc_call(body, out_shape, *args):
    return pl.pallas_call(body, out_shape=out_shape)(*args)


def kernel(x, edge_index, conv1_W, conv1_b, conv2_W, conv2_b, emb_W, emb_b,
           deg_W1, deg_b1, deg_W2, deg_b2, gs_W1, gs_b1, gs_W2, gs_b2,
           fin_W, fin_b):
    n, in_ch = x.shape
    e = edge_index.shape[1]
    hid = conv1_W.shape[1]

    per_w = e // _NW
    nch = -(-per_w // _CE)
    if nch % 2:
        nch += 1
    quota = nch * _CE
    pad = quota - per_w

    row2 = edge_index[0].reshape(_NW, per_w)
    col2 = edge_index[1].reshape(_NW, per_w)
    rowp = jnp.pad(row2, ((0, 0), (0, pad))).reshape(_NW, nch, _CB)
    colp = jnp.pad(col2, ((0, 0), (0, pad)),
                   constant_values=n).reshape(_NW, nch, _CB)

    zeros_n = jnp.zeros((n + _PAD,), jnp.float32)
    zeros_2d = jnp.zeros((n + _PAD, hid), jnp.float32)

    graph_size = n / 100.0
    edge_density = (e / 2.0) / (n * (n - 1) / 2.0)
    avg_path_length = 1.0 / (edge_density + 1e-06)
    stats = jnp.asarray(np.array(
        [[graph_size, edge_density, edge_density, avg_path_length, 1.0]],
        dtype=np.float32))

    deg_k = _make_degree_kernel(n, nch)
    mp_k = _make_mp_kernel(n, nch, hid)

    degc, degr = deg_k(rowp, colp, zeros_n)

    g1, dinv, demb = _tc_call(
        _make_prep_body(float(pad * _NW)),
        (jax.ShapeDtypeStruct((n, hid), jnp.float32),
         jax.ShapeDtypeStruct((n,), jnp.float32),
         jax.ShapeDtypeStruct((1, 8), jnp.float32)),
        degc, degr, x, conv1_W, deg_W1, deg_b1, deg_W2, deg_b2)

    s1 = mp_k(g1, rowp, colp, zeros_2d)

    g2 = _tc_call(
        _mid_body,
        jax.ShapeDtypeStruct((n, hid), jnp.float32),
        s1, g1, dinv, conv1_b, conv2_W)

    s2 = mp_k(g2, rowp, colp, zeros_2d)

    obs = _tc_call(
        _fin_body,
        jax.ShapeDtypeStruct((1, 64), jnp.float32),
        s2, g2, dinv, conv2_b, emb_W, emb_b, demb,
        gs_W1, gs_b1, gs_W2, gs_b2, fin_W, fin_b, stats)

    return obs.reshape(-1)


# speedup vs baseline: 19.3629x; 1.0229x over previous
import functools

import numpy as np
import jax
import jax.numpy as jnp
from jax import lax
from jax.experimental import pallas as pl
from jax.experimental.pallas import tpu as pltpu
from jax.experimental.pallas import tpu_sc as plsc

_NC = 2
_NS = 16
_NW = _NC * _NS
_CB = 256
_CE = _CB
_PAD = 8


def _make_degree_kernel(n, nch):
    npad = n + _PAD
    mesh = plsc.VectorSubcoreMesh(core_axis_name="c", subcore_axis_name="s")

    @functools.partial(
        pl.kernel,
        out_type=(
            jax.ShapeDtypeStruct((_NC, n), jnp.float32),
            jax.ShapeDtypeStruct((_NC, n), jnp.float32),
        ),
        mesh=mesh,
        compiler_params=pltpu.CompilerParams(use_tc_tiling_on_sc=False),
        scratch_types=[
            pltpu.VMEM((nch, _CB), jnp.int32),
            pltpu.VMEM((nch, _CB), jnp.int32),
            pltpu.VMEM((_CB,), jnp.float32),
            pltpu.VMEM_SHARED((npad,), jnp.float32),
            pltpu.VMEM_SHARED((npad,), jnp.float32),
            pltpu.SemaphoreType.DMA,
        ],
    )
    def deg_kernel(row_h, col_h, zeros_h, outc_h, outr_h,
                   ridx, cidx, ones_v, acc_c, acc_r, ssem):
        cid = lax.axis_index("c")
        sid = lax.axis_index("s")
        wid = cid * _NS + sid

        def fill(i, c):
            ones_v[pl.ds(i * 16, 16)] = jnp.ones((16,), jnp.float32)
            return c
        lax.fori_loop(0, _CB // 16, fill, 0)

        pltpu.sync_copy(row_h.at[wid], ridx)
        pltpu.sync_copy(col_h.at[wid], cidx)

        @pl.when(sid == 0)
        def _():
            pltpu.sync_copy(zeros_h, acc_c)
            pltpu.sync_copy(zeros_h, acc_r)
        plsc.subcore_barrier()

        def body(i, c):
            pltpu.async_copy(ones_v, acc_c.at[cidx.at[i]], ssem, add=True)
            pltpu.sync_copy(ones_v, acc_r.at[ridx.at[i]], add=True)
            return c
        lax.fori_loop(0, nch, body, 0)

        def drain(i, c):
            pltpu.make_async_copy(ones_v, acc_c.at[cidx.at[0]], ssem).wait()
            return c
        lax.fori_loop(0, nch, drain, 0)
        plsc.subcore_barrier()

        @pl.when(sid == 0)
        def _():
            pltpu.sync_copy(acc_c.at[pl.ds(0, n)], outc_h.at[cid])
            pltpu.sync_copy(acc_r.at[pl.ds(0, n)], outr_h.at[cid])

    return deg_kernel


def _make_mp_kernel(n, nch, d):
    npad = n + _PAD
    mesh = plsc.VectorSubcoreMesh(core_axis_name="c", subcore_axis_name="s")
    assert nch % 2 == 0

    @functools.partial(
        pl.kernel,
        out_type=jax.ShapeDtypeStruct((_NC, n, d), jnp.float32),
        mesh=mesh,
        compiler_params=pltpu.CompilerParams(use_tc_tiling_on_sc=False),
        scratch_types=[
            pltpu.VMEM((nch, _CB), jnp.int32),
            pltpu.VMEM((nch, _CB), jnp.int32),
            [pltpu.VMEM((_CB, d), jnp.float32) for _ in range(4)],
            pltpu.VMEM_SHARED((npad, d), jnp.float32),
            [pltpu.SemaphoreType.DMA for _ in range(4)],
            [pltpu.SemaphoreType.DMA for _ in range(4)],
        ],
    )
    def mp_kernel(g_h, row_h, col_h, zeros_h, out_h,
                  ridx, cidx, msg, acc, gsem, ssem):
        cid = lax.axis_index("c")
        sid = lax.axis_index("s")
        wid = cid * _NS + sid

        pltpu.sync_copy(row_h.at[wid], ridx)
        pltpu.sync_copy(col_h.at[wid], cidx)

        rpt = n // _NS
        pltpu.sync_copy(zeros_h.at[pl.ds(sid * rpt, rpt)],
                        acc.at[pl.ds(sid * rpt, rpt)])

        @pl.when(sid == 0)
        def _():
            pltpu.sync_copy(zeros_h.at[pl.ds(n, _PAD)], acc.at[pl.ds(n, _PAD)])
        plsc.subcore_barrier()

        def wait_gather(b):
            pltpu.make_async_copy(g_h.at[ridx.at[0]], msg[b], gsem[b]).wait()

        def wait_scatter(b):
            pltpu.make_async_copy(msg[b], acc.at[cidx.at[0]], ssem[b]).wait()

        pltpu.async_copy(g_h.at[ridx.at[0]], msg[0], gsem[0])
        pltpu.async_copy(g_h.at[ridx.at[1]], msg[1], gsem[1])

        assert nch % 4 == 0

        def body(p, c):
            for k in range(4):
                i = 4 * p + k
                b = k
                b2 = (k + 2) % 4
                wait_gather(b)
                pltpu.async_copy(msg[b], acc.at[cidx.at[i]], ssem[b], add=True)

                @pl.when(jnp.logical_and(i >= 2, i + 2 < nch))
                def _():
                    wait_scatter(b2)

                @pl.when(i + 2 < nch)
                def _():
                    pltpu.async_copy(g_h.at[ridx.at[i + 2]], msg[b2], gsem[b2])
            return c
        lax.fori_loop(0, nch // 4, body, 0)
        for j in range(nch - 4, nch):
            wait_scatter(j % 4)
        plsc.subcore_barrier()

        @pl.when(sid == 0)
        def _():
            pltpu.sync_copy(acc.at[pl.ds(0, n)], out_h.at[cid])

    return mp_kernel


def _make_prep_body(pad_corr):
    def _prep_body(degc_ref, degr_ref, x_ref, w1_ref, dw1_ref, db1_ref,
                   dw2_ref, db2_ref, g1_ref, dinv_ref, demb_ref):
        degc = degc_ref[...]
        dinv = lax.rsqrt(degc[0] + degc[1] + 1.0)
        dinv_ref[...] = dinv
        g1_ref[...] = jnp.dot(x_ref[...], w1_ref[...],
                              preferred_element_type=jnp.float32) * dinv[:, None]

        degr = degr_ref[...]
        outdeg = degr[0] + degr[1]
        n = outdeg.shape[0]
        lane = lax.broadcasted_iota(jnp.int32, (n,), 0)
        outdeg = outdeg - jnp.where(lane == 0, pad_corr, 0.0)
        demb = db2_ref[...][None, :]
        for j in range(8):
            sj = jnp.sum(jax.nn.relu(outdeg * dw1_ref[0, j] + db1_ref[j])) / n
            demb = demb + sj * dw2_ref[j:j + 1, :]
        demb_ref[...] = demb
    return _prep_body


def _mid_body(s_ref, g1_ref, dinv_ref, b1_ref, w2_ref, g2_ref):
    dinv = dinv_ref[...]
    h1 = jax.nn.relu((s_ref[0] + s_ref[1] + g1_ref[...]) * dinv[:, None]
                     + b1_ref[...][None, :])
    g2_ref[...] = jnp.dot(h1, w2_ref[...],
                          preferred_element_type=jnp.float32) * dinv[:, None]


def _fin_body(s_ref, g2_ref, dinv_ref, b2_ref, embw_ref, embb_ref, demb_ref,
              gsw1_ref, gsb1_ref, gsw2_ref, gsb2_ref, finw_ref, finb_ref,
              stats_ref, out_ref):
    dinv = dinv_ref[...]
    h2 = jax.nn.relu((s_ref[0] + s_ref[1] + g2_ref[...]) * dinv[:, None]
                     + b2_ref[...][None, :])
    hid = h2.shape[1]
    p_sum = jnp.sum(h2, axis=0, keepdims=True)
    p_mean = p_sum / h2.shape[0]
    p_max = jnp.max(h2, axis=0, keepdims=True)
    embw = embw_ref[...]
    dot = functools.partial(jnp.dot, preferred_element_type=jnp.float32)
    ge = jax.nn.relu(dot(p_mean, embw[0:hid])
                     + dot(p_sum, embw[hid:2 * hid])
                     + dot(p_max, embw[2 * hid:3 * hid])
                     + embb_ref[...][None, :])
    a1 = jax.nn.relu(dot(stats_ref[...], gsw1_ref[...]) + gsb1_ref[...][None, :])
    gse = dot(a1, gsw2_ref[...]) + gsb2_ref[...][None, :]
    finw = finw_ref[...]
    obs = (dot(ge, finw[0:64]) + dot(demb_ref[...], finw[64:72])
           + dot(gse, finw[72:88]) + finb_ref[...][None, :])
    out_ref[...] = obs


def _tc_call(body, out_shape, *args):
    return pl.pallas_call(body, out_shape=out_shape)(*args)


def kernel(x, edge_index, conv1_W, conv1_b, conv2_W, conv2_b, emb_W, emb_b,
           deg_W1, deg_b1, deg_W2, deg_b2, gs_W1, gs_b1, gs_W2, gs_b2,
           fin_W, fin_b):
    n, in_ch = x.shape
    e = edge_index.shape[1]
    hid = conv1_W.shape[1]

    per_w = e // _NW
    nch = -(-per_w // _CE)
    if nch % 2:
        nch += 1
    quota = nch * _CE
    pad = quota - per_w

    row2 = edge_index[0].reshape(_NW, per_w)
    col2 = edge_index[1].reshape(_NW, per_w)
    rowp = jnp.pad(row2, ((0, 0), (0, pad))).reshape(_NW, nch, _CB)
    colp = jnp.pad(col2, ((0, 0), (0, pad)),
                   constant_values=n).reshape(_NW, nch, _CB)

    zeros_n = jnp.zeros((n + _PAD,), jnp.float32)
    zeros_2d = jnp.zeros((n + _PAD, hid), jnp.float32)

    graph_size = n / 100.0
    edge_density = (e / 2.0) / (n * (n - 1) / 2.0)
    avg_path_length = 1.0 / (edge_density + 1e-06)
    stats = jnp.asarray(np.array(
        [[graph_size, edge_density, edge_density, avg_path_length, 1.0]],
        dtype=np.float32))

    deg_k = _make_degree_kernel(n, nch)
    mp_k = _make_mp_kernel(n, nch, hid)

    degc, degr = deg_k(rowp, colp, zeros_n)

    g1, dinv, demb = _tc_call(
        _make_prep_body(float(pad * _NW)),
        (jax.ShapeDtypeStruct((n, hid), jnp.float32),
         jax.ShapeDtypeStruct((n,), jnp.float32),
         jax.ShapeDtypeStruct((1, 8), jnp.float32)),
        degc, degr, x, conv1_W, deg_W1, deg_b1, deg_W2, deg_b2)

    s1 = mp_k(g1, rowp, colp, zeros_2d)

    g2 = _tc_call(
        _mid_body,
        jax.ShapeDtypeStruct((n, hid), jnp.float32),
        s1, g1, dinv, conv1_b, conv2_W)

    s2 = mp_k(g2, rowp, colp, zeros_2d)

    obs = _tc_call(
        _fin_body,
        jax.ShapeDtypeStruct((1, 64), jnp.float32),
        s2, g2, dinv, conv2_b, emb_W, emb_b, demb,
        gs_W1, gs_b1, gs_W2, gs_b2, fin_W, fin_b, stats)

    return obs.reshape(-1)


# speedup vs baseline: 19.4793x; 1.0060x over previous
import functools

import numpy as np
import jax
import jax.numpy as jnp
from jax import lax
from jax.experimental import pallas as pl
from jax.experimental.pallas import tpu as pltpu
from jax.experimental.pallas import tpu_sc as plsc

_NC = 2
_NS = 16
_NW = _NC * _NS
_CB = 256
_CE = _CB
_PAD = 8


def _make_degree_kernel(n, nch):
    npad = n + _PAD
    mesh = plsc.VectorSubcoreMesh(core_axis_name="c", subcore_axis_name="s")

    @functools.partial(
        pl.kernel,
        out_type=(
            jax.ShapeDtypeStruct((_NC, n), jnp.float32),
            jax.ShapeDtypeStruct((_NC, n), jnp.float32),
        ),
        mesh=mesh,
        compiler_params=pltpu.CompilerParams(use_tc_tiling_on_sc=False),
        scratch_types=[
            pltpu.VMEM((nch, _CB), jnp.int32),
            pltpu.VMEM((nch, _CB), jnp.int32),
            pltpu.VMEM((_CB,), jnp.float32),
            pltpu.VMEM_SHARED((npad,), jnp.float32),
            pltpu.VMEM_SHARED((npad,), jnp.float32),
            pltpu.SemaphoreType.DMA,
            pltpu.SemaphoreType.DMA,
        ],
    )
    def deg_kernel(row_h, col_h, zeros_h, outc_h, outr_h,
                   ridx, cidx, ones_v, acc_c, acc_r, ssem, rsem):
        cid = lax.axis_index("c")
        sid = lax.axis_index("s")
        wid = cid * _NS + sid

        def fill(i, c):
            ones_v[pl.ds(i * 16, 16)] = jnp.ones((16,), jnp.float32)
            return c
        lax.fori_loop(0, _CB // 16, fill, 0)

        pltpu.sync_copy(row_h.at[wid], ridx)
        pltpu.sync_copy(col_h.at[wid], cidx)

        @pl.when(sid == 0)
        def _():
            pltpu.sync_copy(zeros_h, acc_c)
            pltpu.sync_copy(zeros_h, acc_r)
        plsc.subcore_barrier()

        def body(i, c):
            pltpu.async_copy(ones_v, acc_c.at[cidx.at[i]], ssem, add=True)
            pltpu.async_copy(ones_v, acc_r.at[ridx.at[i]], rsem, add=True)
            return c
        lax.fori_loop(0, nch, body, 0)

        def drain(i, c):
            pltpu.make_async_copy(ones_v, acc_c.at[cidx.at[0]], ssem).wait()
            pltpu.make_async_copy(ones_v, acc_r.at[ridx.at[0]], rsem).wait()
            return c
        lax.fori_loop(0, nch, drain, 0)
        plsc.subcore_barrier()

        @pl.when(sid == 0)
        def _():
            pltpu.sync_copy(acc_c.at[pl.ds(0, n)], outc_h.at[cid])
            pltpu.sync_copy(acc_r.at[pl.ds(0, n)], outr_h.at[cid])

    return deg_kernel


def _make_mp_kernel(n, nch, d):
    npad = n + _PAD
    mesh = plsc.VectorSubcoreMesh(core_axis_name="c", subcore_axis_name="s")
    assert nch % 2 == 0

    @functools.partial(
        pl.kernel,
        out_type=jax.ShapeDtypeStruct((_NC, n, d), jnp.float32),
        mesh=mesh,
        compiler_params=pltpu.CompilerParams(use_tc_tiling_on_sc=False),
        scratch_types=[
            pltpu.VMEM((nch, _CB), jnp.int32),
            pltpu.VMEM((nch, _CB), jnp.int32),
            [pltpu.VMEM((_CB, d), jnp.float32) for _ in range(4)],
            pltpu.VMEM_SHARED((npad, d), jnp.float32),
            [pltpu.SemaphoreType.DMA for _ in range(4)],
            [pltpu.SemaphoreType.DMA for _ in range(4)],
        ],
    )
    def mp_kernel(g_h, row_h, col_h, zeros_h, out_h,
                  ridx, cidx, msg, acc, gsem, ssem):
        cid = lax.axis_index("c")
        sid = lax.axis_index("s")
        wid = cid * _NS + sid

        pltpu.sync_copy(row_h.at[wid], ridx)
        pltpu.sync_copy(col_h.at[wid], cidx)

        rpt = n // _NS
        pltpu.sync_copy(zeros_h.at[pl.ds(sid * rpt, rpt)],
                        acc.at[pl.ds(sid * rpt, rpt)])

        @pl.when(sid == 0)
        def _():
            pltpu.sync_copy(zeros_h.at[pl.ds(n, _PAD)], acc.at[pl.ds(n, _PAD)])
        plsc.subcore_barrier()

        def wait_gather(b):
            pltpu.make_async_copy(g_h.at[ridx.at[0]], msg[b], gsem[b]).wait()

        def wait_scatter(b):
            pltpu.make_async_copy(msg[b], acc.at[cidx.at[0]], ssem[b]).wait()

        pltpu.async_copy(g_h.at[ridx.at[0]], msg[0], gsem[0])
        pltpu.async_copy(g_h.at[ridx.at[1]], msg[1], gsem[1])

        assert nch % 4 == 0

        def body(p, c):
            for k in range(4):
                i = 4 * p + k
                b = k
                b2 = (k + 2) % 4
                wait_gather(b)
                pltpu.async_copy(msg[b], acc.at[cidx.at[i]], ssem[b], add=True)

                @pl.when(jnp.logical_and(i >= 2, i + 2 < nch))
                def _():
                    wait_scatter(b2)

                @pl.when(i + 2 < nch)
                def _():
                    pltpu.async_copy(g_h.at[ridx.at[i + 2]], msg[b2], gsem[b2])
            return c
        lax.fori_loop(0, nch // 4, body, 0)
        for j in range(nch - 4, nch):
            wait_scatter(j % 4)
        plsc.subcore_barrier()

        @pl.when(sid == 0)
        def _():
            pltpu.sync_copy(acc.at[pl.ds(0, n)], out_h.at[cid])

    return mp_kernel


def _mm_body(x_ref, w1_ref, p_ref):
    p_ref[...] = jnp.dot(x_ref[...], w1_ref[...],
                         preferred_element_type=jnp.float32)


def _make_prep_body(pad_corr):
    def _prep_body(degc_ref, degr_ref, p_ref, dw1_ref, db1_ref,
                   dw2_ref, db2_ref, g1_ref, dinv_ref, demb_ref):
        degc = degc_ref[...]
        dinv = lax.rsqrt(degc[0] + degc[1] + 1.0)
        dinv_ref[...] = dinv
        g1_ref[...] = p_ref[...] * dinv[:, None]

        degr = degr_ref[...]
        outdeg = degr[0] + degr[1]
        n = outdeg.shape[0]
        lane = lax.broadcasted_iota(jnp.int32, (n,), 0)
        outdeg = outdeg - jnp.where(lane == 0, pad_corr, 0.0)
        demb = db2_ref[...][None, :]
        for j in range(8):
            sj = jnp.sum(jax.nn.relu(outdeg * dw1_ref[0, j] + db1_ref[j])) / n
            demb = demb + sj * dw2_ref[j:j + 1, :]
        demb_ref[...] = demb
    return _prep_body


def _mid_body(s_ref, g1_ref, dinv_ref, b1_ref, w2_ref, g2_ref):
    dinv = dinv_ref[...]
    h1 = jax.nn.relu((s_ref[0] + s_ref[1] + g1_ref[...]) * dinv[:, None]
                     + b1_ref[...][None, :])
    g2_ref[...] = jnp.dot(h1, w2_ref[...],
                          preferred_element_type=jnp.float32) * dinv[:, None]


def _fin_body(s_ref, g2_ref, dinv_ref, b2_ref, embw_ref, embb_ref, demb_ref,
              gsw1_ref, gsb1_ref, gsw2_ref, gsb2_ref, finw_ref, finb_ref,
              stats_ref, out_ref):
    dinv = dinv_ref[...]
    h2 = jax.nn.relu((s_ref[0] + s_ref[1] + g2_ref[...]) * dinv[:, None]
                     + b2_ref[...][None, :])
    hid = h2.shape[1]
    p_sum = jnp.sum(h2, axis=0, keepdims=True)
    p_mean = p_sum / h2.shape[0]
    p_max = jnp.max(h2, axis=0, keepdims=True)
    embw = embw_ref[...]
    dot = functools.partial(jnp.dot, preferred_element_type=jnp.float32)
    ge = jax.nn.relu(dot(p_mean, embw[0:hid])
                     + dot(p_sum, embw[hid:2 * hid])
                     + dot(p_max, embw[2 * hid:3 * hid])
                     + embb_ref[...][None, :])
    a1 = jax.nn.relu(dot(stats_ref[...], gsw1_ref[...]) + gsb1_ref[...][None, :])
    gse = dot(a1, gsw2_ref[...]) + gsb2_ref[...][None, :]
    finw = finw_ref[...]
    obs = (dot(ge, finw[0:64]) + dot(demb_ref[...], finw[64:72])
           + dot(gse, finw[72:88]) + finb_ref[...][None, :])
    out_ref[...] = obs


def _tc_call(body, out_shape, *args):
    return pl.pallas_call(body, out_shape=out_shape)(*args)


def kernel(x, edge_index, conv1_W, conv1_b, conv2_W, conv2_b, emb_W, emb_b,
           deg_W1, deg_b1, deg_W2, deg_b2, gs_W1, gs_b1, gs_W2, gs_b2,
           fin_W, fin_b):
    n, in_ch = x.shape
    e = edge_index.shape[1]
    hid = conv1_W.shape[1]

    per_w = e // _NW
    nch = -(-per_w // _CE)
    if nch % 2:
        nch += 1
    quota = nch * _CE
    pad = quota - per_w

    row2 = edge_index[0].reshape(_NW, per_w)
    col2 = edge_index[1].reshape(_NW, per_w)
    rowp = jnp.pad(row2, ((0, 0), (0, pad))).reshape(_NW, nch, _CB)
    colp = jnp.pad(col2, ((0, 0), (0, pad)),
                   constant_values=n).reshape(_NW, nch, _CB)

    zeros_n = jnp.zeros((n + _PAD,), jnp.float32)
    zeros_2d = jnp.zeros((n + _PAD, hid), jnp.float32)

    graph_size = n / 100.0
    edge_density = (e / 2.0) / (n * (n - 1) / 2.0)
    avg_path_length = 1.0 / (edge_density + 1e-06)
    stats = jnp.asarray(np.array(
        [[graph_size, edge_density, edge_density, avg_path_length, 1.0]],
        dtype=np.float32))

    deg_k = _make_degree_kernel(n, nch)
    mp_k = _make_mp_kernel(n, nch, hid)

    degc, degr = deg_k(rowp, colp, zeros_n)

    p1 = _tc_call(_mm_body, jax.ShapeDtypeStruct((n, hid), jnp.float32),
                  x, conv1_W)

    g1, dinv, demb = _tc_call(
        _make_prep_body(float(pad * _NW)),
        (jax.ShapeDtypeStruct((n, hid), jnp.float32),
         jax.ShapeDtypeStruct((n,), jnp.float32),
         jax.ShapeDtypeStruct((1, 8), jnp.float32)),
        degc, degr, p1, deg_W1, deg_b1, deg_W2, deg_b2)

    s1 = mp_k(g1, rowp, colp, zeros_2d)

    g2 = _tc_call(
        _mid_body,
        jax.ShapeDtypeStruct((n, hid), jnp.float32),
        s1, g1, dinv, conv1_b, conv2_W)

    s2 = mp_k(g2, rowp, colp, zeros_2d)

    obs = _tc_call(
        _fin_body,
        jax.ShapeDtypeStruct((1, 64), jnp.float32),
        s2, g2, dinv, conv2_b, emb_W, emb_b, demb,
        gs_W1, gs_b1, gs_W2, gs_b2, fin_W, fin_b, stats)

    return obs.reshape(-1)


# speedup vs baseline: 19.5634x; 1.0043x over previous
import functools

import numpy as np
import jax
import jax.numpy as jnp
from jax import lax
from jax.experimental import pallas as pl
from jax.experimental.pallas import tpu as pltpu
from jax.experimental.pallas import tpu_sc as plsc

_NC = 2
_NS = 16
_NW = _NC * _NS
_CB = 256
_CE = _CB
_PAD = 8


def _make_degree_kernel(n, nch):
    npad = n + _PAD
    mesh = plsc.VectorSubcoreMesh(core_axis_name="c", subcore_axis_name="s")

    @functools.partial(
        pl.kernel,
        out_type=(
            jax.ShapeDtypeStruct((_NC, n), jnp.float32),
            jax.ShapeDtypeStruct((_NC, n), jnp.float32),
        ),
        mesh=mesh,
        compiler_params=pltpu.CompilerParams(use_tc_tiling_on_sc=False),
        scratch_types=[
            pltpu.VMEM((nch * _CB,), jnp.int32),
            pltpu.VMEM((nch * _CB,), jnp.int32),
            pltpu.VMEM((_CB,), jnp.float32),
            pltpu.VMEM_SHARED((npad,), jnp.float32),
            pltpu.VMEM_SHARED((npad,), jnp.float32),
            pltpu.SemaphoreType.DMA,
            pltpu.SemaphoreType.DMA,
        ],
    )
    def deg_kernel(row_h, col_h, zeros_h, outc_h, outr_h,
                   ridx, cidx, ones_v, acc_c, acc_r, ssem, rsem):
        cid = lax.axis_index("c")
        sid = lax.axis_index("s")
        wid = cid * _NS + sid

        def fill(i, c):
            ones_v[pl.ds(i * 16, 16)] = jnp.ones((16,), jnp.float32)
            return c
        lax.fori_loop(0, _CB // 16, fill, 0)

        per_w = row_h.shape[1]
        pltpu.sync_copy(row_h.at[wid], ridx.at[pl.ds(0, per_w)])
        pltpu.sync_copy(col_h.at[wid], cidx.at[pl.ds(0, per_w)])
        for k in range((nch * _CB - per_w) // 16):
            ridx[pl.ds(per_w + k * 16, 16)] = jnp.zeros((16,), jnp.int32)
            cidx[pl.ds(per_w + k * 16, 16)] = jnp.full((16,), n, jnp.int32)

        @pl.when(sid == 0)
        def _():
            pltpu.sync_copy(zeros_h, acc_c)
            pltpu.sync_copy(zeros_h, acc_r)
        plsc.subcore_barrier()

        def body(i, c):
            pltpu.async_copy(ones_v, acc_c.at[cidx.at[pl.ds(i * _CB, _CB)]],
                             ssem, add=True)
            pltpu.async_copy(ones_v, acc_r.at[ridx.at[pl.ds(i * _CB, _CB)]],
                             rsem, add=True)
            return c
        lax.fori_loop(0, nch, body, 0)

        def drain(i, c):
            pltpu.make_async_copy(ones_v, acc_c.at[cidx.at[pl.ds(0, _CB)]],
                                  ssem).wait()
            pltpu.make_async_copy(ones_v, acc_r.at[ridx.at[pl.ds(0, _CB)]],
                                  rsem).wait()
            return c
        lax.fori_loop(0, nch, drain, 0)
        plsc.subcore_barrier()

        @pl.when(sid == 0)
        def _():
            pltpu.sync_copy(acc_c.at[pl.ds(0, n)], outc_h.at[cid])
            pltpu.sync_copy(acc_r.at[pl.ds(0, n)], outr_h.at[cid])

    return deg_kernel


def _make_mp_kernel(n, nch, d):
    npad = n + _PAD
    mesh = plsc.VectorSubcoreMesh(core_axis_name="c", subcore_axis_name="s")
    assert nch % 2 == 0

    @functools.partial(
        pl.kernel,
        out_type=jax.ShapeDtypeStruct((_NC, n, d), jnp.float32),
        mesh=mesh,
        compiler_params=pltpu.CompilerParams(use_tc_tiling_on_sc=False),
        scratch_types=[
            pltpu.VMEM((nch * _CB,), jnp.int32),
            pltpu.VMEM((nch * _CB,), jnp.int32),
            [pltpu.VMEM((_CB, d), jnp.float32) for _ in range(4)],
            pltpu.VMEM_SHARED((npad, d), jnp.float32),
            [pltpu.SemaphoreType.DMA for _ in range(4)],
            [pltpu.SemaphoreType.DMA for _ in range(4)],
        ],
    )
    def mp_kernel(g_h, row_h, col_h, zeros_h, out_h,
                  ridx, cidx, msg, acc, gsem, ssem):
        cid = lax.axis_index("c")
        sid = lax.axis_index("s")
        wid = cid * _NS + sid

        per_w = row_h.shape[1]
        pltpu.sync_copy(row_h.at[wid], ridx.at[pl.ds(0, per_w)])
        pltpu.sync_copy(col_h.at[wid], cidx.at[pl.ds(0, per_w)])
        for k in range((nch * _CB - per_w) // 16):
            ridx[pl.ds(per_w + k * 16, 16)] = jnp.zeros((16,), jnp.int32)
            cidx[pl.ds(per_w + k * 16, 16)] = jnp.full((16,), n, jnp.int32)

        rpt = n // _NS
        pltpu.sync_copy(zeros_h.at[pl.ds(sid * rpt, rpt)],
                        acc.at[pl.ds(sid * rpt, rpt)])

        @pl.when(sid == 0)
        def _():
            pltpu.sync_copy(zeros_h.at[pl.ds(n, _PAD)], acc.at[pl.ds(n, _PAD)])
        plsc.subcore_barrier()

        def wait_gather(b):
            pltpu.make_async_copy(g_h.at[ridx.at[pl.ds(0, _CB)]], msg[b], gsem[b]).wait()

        def wait_scatter(b):
            pltpu.make_async_copy(msg[b], acc.at[cidx.at[pl.ds(0, _CB)]], ssem[b]).wait()

        pltpu.async_copy(g_h.at[ridx.at[pl.ds(0, _CB)]], msg[0], gsem[0])
        pltpu.async_copy(g_h.at[ridx.at[pl.ds(_CB, _CB)]], msg[1], gsem[1])

        assert nch % 4 == 0

        def body(p, c):
            for k in range(4):
                i = 4 * p + k
                b = k
                b2 = (k + 2) % 4
                wait_gather(b)
                pltpu.async_copy(msg[b], acc.at[cidx.at[pl.ds(i * _CB, _CB)]], ssem[b], add=True)

                @pl.when(jnp.logical_and(i >= 2, i + 2 < nch))
                def _():
                    wait_scatter(b2)

                @pl.when(i + 2 < nch)
                def _():
                    pltpu.async_copy(g_h.at[ridx.at[pl.ds((i + 2) * _CB, _CB)]], msg[b2], gsem[b2])
            return c
        lax.fori_loop(0, nch // 4, body, 0)
        for j in range(nch - 4, nch):
            wait_scatter(j % 4)
        plsc.subcore_barrier()

        @pl.when(sid == 0)
        def _():
            pltpu.sync_copy(acc.at[pl.ds(0, n)], out_h.at[cid])

    return mp_kernel


def _mm_body(x_ref, w1_ref, p_ref):
    p_ref[...] = jnp.dot(x_ref[...], w1_ref[...],
                         preferred_element_type=jnp.float32)


def _make_prep_body(pad_corr):
    def _prep_body(degc_ref, degr_ref, p_ref, dw1_ref, db1_ref,
                   dw2_ref, db2_ref, g1_ref, dinv_ref, demb_ref):
        degc = degc_ref[...]
        dinv = lax.rsqrt(degc[0] + degc[1] + 1.0)
        dinv_ref[...] = dinv
        g1_ref[...] = p_ref[...] * dinv[:, None]

        degr = degr_ref[...]
        outdeg = degr[0] + degr[1]
        n = outdeg.shape[0]
        lane = lax.broadcasted_iota(jnp.int32, (n,), 0)
        outdeg = outdeg - jnp.where(lane == 0, pad_corr, 0.0)
        demb = db2_ref[...][None, :]
        for j in range(8):
            sj = jnp.sum(jax.nn.relu(outdeg * dw1_ref[0, j] + db1_ref[j])) / n
            demb = demb + sj * dw2_ref[j:j + 1, :]
        demb_ref[...] = demb
    return _prep_body


def _mid_body(s_ref, g1_ref, dinv_ref, b1_ref, w2_ref, g2_ref):
    dinv = dinv_ref[...]
    h1 = jax.nn.relu((s_ref[0] + s_ref[1] + g1_ref[...]) * dinv[:, None]
                     + b1_ref[...][None, :])
    g2_ref[...] = jnp.dot(h1, w2_ref[...],
                          preferred_element_type=jnp.float32) * dinv[:, None]


def _fin_body(s_ref, g2_ref, dinv_ref, b2_ref, embw_ref, embb_ref, demb_ref,
              gsw1_ref, gsb1_ref, gsw2_ref, gsb2_ref, finw_ref, finb_ref,
              stats_ref, out_ref):
    dinv = dinv_ref[...]
    h2 = jax.nn.relu((s_ref[0] + s_ref[1] + g2_ref[...]) * dinv[:, None]
                     + b2_ref[...][None, :])
    hid = h2.shape[1]
    p_sum = jnp.sum(h2, axis=0, keepdims=True)
    p_mean = p_sum / h2.shape[0]
    p_max = jnp.max(h2, axis=0, keepdims=True)
    embw = embw_ref[...]
    dot = functools.partial(jnp.dot, preferred_element_type=jnp.float32)
    ge = jax.nn.relu(dot(p_mean, embw[0:hid])
                     + dot(p_sum, embw[hid:2 * hid])
                     + dot(p_max, embw[2 * hid:3 * hid])
                     + embb_ref[...][None, :])
    a1 = jax.nn.relu(dot(stats_ref[...], gsw1_ref[...]) + gsb1_ref[...][None, :])
    gse = dot(a1, gsw2_ref[...]) + gsb2_ref[...][None, :]
    finw = finw_ref[...]
    obs = (dot(ge, finw[0:64]) + dot(demb_ref[...], finw[64:72])
           + dot(gse, finw[72:88]) + finb_ref[...][None, :])
    out_ref[...] = obs


def _tc_call(body, out_shape, *args):
    return pl.pallas_call(body, out_shape=out_shape)(*args)


def kernel(x, edge_index, conv1_W, conv1_b, conv2_W, conv2_b, emb_W, emb_b,
           deg_W1, deg_b1, deg_W2, deg_b2, gs_W1, gs_b1, gs_W2, gs_b2,
           fin_W, fin_b):
    n, in_ch = x.shape
    e = edge_index.shape[1]
    hid = conv1_W.shape[1]

    per_w = e // _NW
    nch = -(-per_w // _CE)
    if nch % 2:
        nch += 1
    quota = nch * _CE
    pad = quota - per_w

    rowp = edge_index[0].reshape(_NW, per_w)
    colp = edge_index[1].reshape(_NW, per_w)

    zeros_n = jnp.zeros((n + _PAD,), jnp.float32)
    zeros_2d = jnp.zeros((n + _PAD, hid), jnp.float32)

    graph_size = n / 100.0
    edge_density = (e / 2.0) / (n * (n - 1) / 2.0)
    avg_path_length = 1.0 / (edge_density + 1e-06)
    stats = jnp.asarray(np.array(
        [[graph_size, edge_density, edge_density, avg_path_length, 1.0]],
        dtype=np.float32))

    deg_k = _make_degree_kernel(n, nch)
    mp_k = _make_mp_kernel(n, nch, hid)

    degc, degr = deg_k(rowp, colp, zeros_n)

    p1 = _tc_call(_mm_body, jax.ShapeDtypeStruct((n, hid), jnp.float32),
                  x, conv1_W)

    g1, dinv, demb = _tc_call(
        _make_prep_body(float(pad * _NW)),
        (jax.ShapeDtypeStruct((n, hid), jnp.float32),
         jax.ShapeDtypeStruct((n,), jnp.float32),
         jax.ShapeDtypeStruct((1, 8), jnp.float32)),
        degc, degr, p1, deg_W1, deg_b1, deg_W2, deg_b2)

    s1 = mp_k(g1, rowp, colp, zeros_2d)

    g2 = _tc_call(
        _mid_body,
        jax.ShapeDtypeStruct((n, hid), jnp.float32),
        s1, g1, dinv, conv1_b, conv2_W)

    s2 = mp_k(g2, rowp, colp, zeros_2d)

    obs = _tc_call(
        _fin_body,
        jax.ShapeDtypeStruct((1, 64), jnp.float32),
        s2, g2, dinv, conv2_b, emb_W, emb_b, demb,
        gs_W1, gs_b1, gs_W2, gs_b2, fin_W, fin_b, stats)

    return obs.reshape(-1)
